# baseline (speedup 1.0000x reference)
import functools

import jax
import jax.numpy as jnp
from jax import lax
from jax.experimental import pallas as pl
from jax.experimental.pallas import tpu as pltpu
from jax.experimental.pallas import tpu_sc as plsc

D = 32
B = 16384
L = 16

_info = plsc.get_sparse_core_info()
NC = _info.num_cores
NS = _info.num_subcores
NW = NC * NS
BPW = B // NW
GROUPS = BPW // L

MARGIN = 1.0
EPS = 1e-6


def _sqrt16(x):
    x = jnp.maximum(x, jnp.float32(1e-30))
    i = lax.bitcast_convert_type(x, jnp.int32)
    i = jnp.int32(0x1FBD1DF5) + lax.shift_right_arithmetic(i, 1)
    y = lax.bitcast_convert_type(i, jnp.float32)
    for _ in range(3):
        y = jnp.float32(0.5) * (y + x / y)
    return y


def _tec_body(a_hbm, p_hbm, n_hbm, w_hbm, out_hbm,
              idx_a, idx_p, idx_n, ea_v, ep_v, en_v, part_v, sem):
    wid = lax.axis_index("s") * NC + lax.axis_index("c")
    base = wid * BPW

    pltpu.sync_copy(a_hbm.at[pl.ds(base, BPW)], idx_a)
    pltpu.sync_copy(p_hbm.at[pl.ds(base * 3, BPW * 3)], idx_p)
    pltpu.sync_copy(n_hbm.at[pl.ds(base * 3, BPW * 3)], idx_n)
    cp_a = pltpu.async_copy(w_hbm.at[idx_a], ea_v, sem)
    cp_p = pltpu.async_copy(w_hbm.at[idx_p], ep_v, sem)
    cp_n = pltpu.async_copy(w_hbm.at[idx_n], en_v, sem)
    cp_a.wait()
    cp_p.wait()
    cp_n.wait()

    lanes = lax.iota(jnp.int32, L)

    def group(g, loss_vec):
        rows_a = g * L + lanes
        rows3 = rows_a * 3
        xrefs = (ep_v, ep_v, ep_v, en_v, en_v, en_v)
        xrows = (rows3, rows3 + 1, rows3 + 2, rows3, rows3 + 1, rows3 + 2)
        acc = [jnp.zeros((L,), jnp.float32) for _ in range(6)]
        for d in range(D):
            col = jnp.full((L,), d, jnp.int32)
            ea_d = plsc.load_gather(ea_v, [rows_a, col]) + jnp.float32(EPS)
            for j in range(6):
                t = ea_d - plsc.load_gather(xrefs[j], [xrows[j], col])
                acc[j] = acc[j] + t * t
        dist = [_sqrt16(acc[j]) for j in range(6)]
        for k in range(5):
            loss_vec = loss_vec + jnp.maximum(
                dist[k] - dist[k + 1] + jnp.float32(MARGIN), jnp.float32(0.0))
        return loss_vec

    loss_vec = lax.fori_loop(0, GROUPS, group, jnp.zeros((L,), jnp.float32))
    part_v[...] = loss_vec
    pltpu.sync_copy(part_v, out_hbm.at[wid])


@functools.partial(jax.jit, static_argnums=())
def _partial_losses(a, p_flat, n_flat, w):
    mesh = plsc.VectorSubcoreMesh(core_axis_name="c", subcore_axis_name="s")
    f = pl.kernel(
        _tec_body,
        mesh=mesh,
        compiler_params=pltpu.CompilerParams(
            needs_layout_passes=False, use_tc_tiling_on_sc=False),
        out_type=jax.ShapeDtypeStruct((NW, L), jnp.float32),
        scratch_types=[
            pltpu.VMEM((BPW,), jnp.int32),
            pltpu.VMEM((BPW * 3,), jnp.int32),
            pltpu.VMEM((BPW * 3,), jnp.int32),
            pltpu.VMEM((BPW, D), jnp.float32),
            pltpu.VMEM((BPW * 3, D), jnp.float32),
            pltpu.VMEM((BPW * 3, D), jnp.float32),
            pltpu.VMEM((L,), jnp.float32),
            pltpu.SemaphoreType.DMA,
        ],
    )
    return f(a, p_flat, n_flat, w)


def kernel(a, p, n, W):
    parts = _partial_losses(a, p.reshape(-1), n.reshape(-1), W)
    return jnp.sum(parts) / jnp.float32(B)
